# Initial kernel scaffold; baseline (speedup 1.0000x reference)
import jax
import jax.numpy as jnp
from jax.experimental import pallas as pl


def kernel(x, embeddings):
    raise NotImplementedError("write your pallas kernel here")


# speedup vs baseline: 1.3513x; 1.3513x over previous
import functools

import jax
import jax.numpy as jnp
from jax.experimental import pallas as pl

_N_EMB = 1024
_EMB_DIM = 256
_BETA = 0.25
_BLK = 512


def _vq_body(fx_ref, e_ref, esq_ref, xsq_ref, emb_ref, loss_ref):
    fx = fx_ref[...]
    emb_tab = e_ref[...]
    esq = esq_ref[...]
    xsq = xsq_ref[...]

    s = jax.lax.dot_general(
        fx, emb_tab, (((1,), (1,)), ((), ())),
        preferred_element_type=jnp.float32)
    dist = (xsq + esq) - 2.0 * s

    mind = jnp.min(dist, axis=-1, keepdims=True)
    iota = jax.lax.broadcasted_iota(jnp.int32, dist.shape, 1)
    idx = jnp.min(jnp.where(dist == mind, iota, _N_EMB),
                  axis=-1, keepdims=True)
    onehot = (iota == idx).astype(jnp.float32)

    gathered = jax.lax.dot_general(
        onehot, emb_tab, (((1,), (0,)), ((), ())),
        preferred_element_type=jnp.float32)
    emb_ref[...] = (gathered + fx) - fx

    @pl.when(pl.program_id(0) == 0)
    def _init():
        loss_ref[...] = jnp.zeros_like(loss_ref)

    loss_ref[...] += jnp.sum(mind, axis=(0, 1), keepdims=True)


@functools.partial(jax.jit, static_argnames=())
def _vq(fx, embeddings, esq, xsq):
    n = fx.shape[0]
    grid = n // _BLK
    return pl.pallas_call(
        _vq_body,
        grid=(grid,),
        in_specs=[
            pl.BlockSpec((_BLK, _EMB_DIM), lambda i: (i, 0)),
            pl.BlockSpec((_N_EMB, _EMB_DIM), lambda i: (0, 0)),
            pl.BlockSpec((1, _N_EMB), lambda i: (0, 0)),
            pl.BlockSpec((_BLK, 1), lambda i: (i, 0)),
        ],
        out_specs=[
            pl.BlockSpec((_BLK, _EMB_DIM), lambda i: (i, 0)),
            pl.BlockSpec((1, 1), lambda i: (0, 0)),
        ],
        out_shape=[
            jax.ShapeDtypeStruct((n, _EMB_DIM), jnp.float32),
            jax.ShapeDtypeStruct((1, 1), jnp.float32),
        ],
    )(fx, embeddings, esq, xsq)


def kernel(x, embeddings):
    b, c, h, w = x.shape
    fx = jnp.transpose(x, (0, 2, 3, 1)).reshape(b * h * w, c)
    xsq = jnp.sum(fx ** 2, axis=-1, keepdims=True)
    esq = jnp.sum(embeddings ** 2, axis=-1)[None, :]
    emb_flat, loss_sum = _vq(fx, embeddings, esq, xsq)
    emb = jnp.transpose(emb_flat.reshape(b, h, w, c), (0, 3, 1, 2))
    loss = loss_sum[0, 0] * ((1.0 + _BETA) / (b * c * h * w))
    return emb, loss


# speedup vs baseline: 1.4680x; 1.0863x over previous
import functools

import jax
import jax.numpy as jnp
from jax.experimental import pallas as pl

_N_EMB = 1024
_EMB_DIM = 256
_BETA = 0.25


def _vq_body(xb_ref, e_ref, esq_ref, emb_ref, loss_ref):
    xb = xb_ref[0]
    emb_tab = e_ref[...]
    esq = esq_ref[...]

    s = jax.lax.dot_general(
        emb_tab, xb, (((1,), (0,)), ((), ())),
        preferred_element_type=jnp.float32)
    xsq = jnp.sum(xb * xb, axis=0, keepdims=True)
    dist = (xsq + esq) - 2.0 * s

    mind = jnp.min(dist, axis=0, keepdims=True)
    iota = jax.lax.broadcasted_iota(jnp.int32, dist.shape, 0)
    idx = jnp.min(jnp.where(dist == mind, iota, _N_EMB),
                  axis=0, keepdims=True)
    onehot = (iota == idx).astype(jnp.float32)

    gathered = jax.lax.dot_general(
        emb_tab, onehot, (((0,), (0,)), ((), ())),
        preferred_element_type=jnp.float32)
    emb_ref[0] = (gathered + xb) - xb

    @pl.when(pl.program_id(0) == 0)
    def _init():
        loss_ref[...] = jnp.zeros_like(loss_ref)

    loss_ref[...] += jnp.sum(mind, axis=(0, 1), keepdims=True)


@jax.jit
def _vq(xr, embeddings, esq):
    b = xr.shape[0]
    return pl.pallas_call(
        _vq_body,
        grid=(b,),
        in_specs=[
            pl.BlockSpec((1, _EMB_DIM, 1024), lambda i: (i, 0, 0)),
            pl.BlockSpec((_N_EMB, _EMB_DIM), lambda i: (0, 0)),
            pl.BlockSpec((_N_EMB, 1), lambda i: (0, 0)),
        ],
        out_specs=[
            pl.BlockSpec((1, _EMB_DIM, 1024), lambda i: (i, 0, 0)),
            pl.BlockSpec((1, 1), lambda i: (0, 0)),
        ],
        out_shape=[
            jax.ShapeDtypeStruct((b, _EMB_DIM, 1024), jnp.float32),
            jax.ShapeDtypeStruct((1, 1), jnp.float32),
        ],
    )(xr, embeddings, esq)


def kernel(x, embeddings):
    b, c, h, w = x.shape
    xr = x.reshape(b, c, h * w)
    esq = jnp.sum(embeddings ** 2, axis=-1)[:, None]
    emb_r, loss_sum = _vq(xr, embeddings, esq)
    emb = emb_r.reshape(b, c, h, w)
    loss = loss_sum[0, 0] * ((1.0 + _BETA) / (b * c * h * w))
    return emb, loss
